# baseline (speedup 1.0000x reference)
import functools

import jax
import jax.numpy as jnp
from jax.experimental import pallas as pl
from jax.experimental.pallas import tpu as pltpu


def _monotone_key(x):
    i = jax.lax.bitcast_convert_type(x, jnp.int32)
    flip = jax.lax.shift_right_arithmetic(i, 31) & jnp.int32(0x7FFFFFFF)
    return i ^ flip


def _fused_kernel(x_ref, w1t_ref, w2t_ref, w3t_ref, wct_ref, o_ref):
    xb = x_ref[...]
    R, Q = xb.shape

    h = jnp.maximum(jnp.dot(xb, w1t_ref[...], preferred_element_type=jnp.float32), 0.0)
    h = jnp.maximum(jnp.dot(h, w2t_ref[...], preferred_element_type=jnp.float32), 0.0)
    scores = jnp.dot(h, w3t_ref[...], preferred_element_type=jnp.float32)

    k = (jnp.argmax(scores, axis=-1).astype(jnp.int32) + 1)[:, None]

    keys = _monotone_key(xb)

    lo0 = jnp.full((R, 1), jnp.int32(-2147483648))
    hi0 = jnp.full((R, 1), jnp.int32(2147483647))

    def body(_, carry):
        lo, hi = carry
        mid = (jax.lax.shift_right_arithmetic(lo, 1)
               + jax.lax.shift_right_arithmetic(hi, 1)
               + (lo & hi & jnp.int32(1)))
        cnt = jnp.sum((keys >= mid).astype(jnp.int32), axis=-1, keepdims=True)
        pred = cnt >= k
        lo = jnp.where(pred, mid, lo)
        hi = jnp.where(pred, hi, mid)
        return lo, hi

    carry = (lo0, hi0)
    for _ in range(32):
        carry = body(0, carry)
    lo, hi = carry
    t = lo

    gt = keys > t
    eq = keys == t
    g = jnp.sum(gt.astype(jnp.int32), axis=-1, keepdims=True)

    eq_f = eq.astype(jnp.float32)
    jj = jax.lax.broadcasted_iota(jnp.int32, (Q, Q), 0)
    ii = jax.lax.broadcasted_iota(jnp.int32, (Q, Q), 1)
    strict_lower = (jj < ii).astype(jnp.float32)
    prefix = jnp.dot(eq_f, strict_lower, preferred_element_type=jnp.float32)
    prefix = prefix.astype(jnp.int32)

    mask = gt | (eq & ((g + prefix) < k))
    o_ref[...] = jnp.dot(mask.astype(jnp.float32), wct_ref[...],
                         preferred_element_type=jnp.float32)


@jax.jit
def kernel(x, W1, W2, W3, Wc):
    B, Q = x.shape
    D = Wc.shape[0]
    R = 2048

    w1t = W1.T
    w2t = W2.T
    w3t = W3.T
    wct = Wc.T

    out = pl.pallas_call(
        _fused_kernel,
        grid=(B // R,),
        in_specs=[
            pl.BlockSpec((R, Q), lambda i: (i, 0)),
            pl.BlockSpec((Q, 2 * Q), lambda i: (0, 0)),
            pl.BlockSpec((2 * Q, Q), lambda i: (0, 0)),
            pl.BlockSpec((Q, Q), lambda i: (0, 0)),
            pl.BlockSpec((Q, D), lambda i: (0, 0)),
        ],
        out_specs=pl.BlockSpec((R, D), lambda i: (i, 0)),
        out_shape=jax.ShapeDtypeStruct((B, D), jnp.float32),
        compiler_params=pltpu.CompilerParams(
            dimension_semantics=("parallel",),
        ),
    )(x, w1t, w2t, w3t, wct)
    return out


# speedup vs baseline: 1.2742x; 1.2742x over previous
import functools

import jax
import jax.numpy as jnp
from jax.experimental import pallas as pl
from jax.experimental.pallas import tpu as pltpu


def _monotone_key(x):
    i = jax.lax.bitcast_convert_type(x, jnp.int32)
    flip = jax.lax.shift_right_arithmetic(i, 31) & jnp.int32(0x7FFFFFFF)
    return i ^ flip


def _fused_kernel(x_ref, w1t_ref, w2t_ref, w3t_ref, wct_ref, o_ref):
    xb = x_ref[...]
    R, Q = xb.shape

    h = jnp.maximum(jnp.dot(xb, w1t_ref[...], preferred_element_type=jnp.float32), 0.0)
    h = jnp.maximum(jnp.dot(h, w2t_ref[...], preferred_element_type=jnp.float32), 0.0)
    scores = jnp.dot(h, w3t_ref[...], preferred_element_type=jnp.float32)

    k = (jnp.argmax(scores, axis=-1).astype(jnp.int32) + 1)[:, None]

    keys = _monotone_key(xb)

    lo0 = jnp.full((R, 1), jnp.int32(-2147483648))
    hi0 = jnp.full((R, 1), jnp.int32(2147483647))

    def body(_, carry):
        lo, hi = carry
        mid = (jax.lax.shift_right_arithmetic(lo, 1)
               + jax.lax.shift_right_arithmetic(hi, 1)
               + (lo & hi & jnp.int32(1)))
        cnt = jnp.sum((keys >= mid).astype(jnp.int32), axis=-1, keepdims=True)
        pred = cnt >= k
        lo = jnp.where(pred, mid, lo)
        hi = jnp.where(pred, hi, mid)
        return lo, hi

    carry = (lo0, hi0)
    for _ in range(32):
        carry = body(0, carry)
    lo, hi = carry
    t = lo

    gt = keys > t
    eq = keys == t
    g = jnp.sum(gt.astype(jnp.int32), axis=-1, keepdims=True)

    eq_f = eq.astype(jnp.float32)
    jj = jax.lax.broadcasted_iota(jnp.int32, (Q, Q), 0)
    ii = jax.lax.broadcasted_iota(jnp.int32, (Q, Q), 1)
    strict_lower = (jj < ii).astype(jnp.float32)
    prefix = jnp.dot(eq_f, strict_lower, preferred_element_type=jnp.float32)
    prefix = prefix.astype(jnp.int32)

    mask = gt | (eq & ((g + prefix) < k))
    o_ref[...] = jnp.dot(mask.astype(jnp.float32), wct_ref[...],
                         preferred_element_type=jnp.float32)


@jax.jit
def kernel(x, W1, W2, W3, Wc):
    B, Q = x.shape
    D = Wc.shape[0]
    R = 1024

    w1t = W1.T
    w2t = W2.T
    w3t = W3.T
    wct = Wc.T

    out = pl.pallas_call(
        _fused_kernel,
        grid=(B // R,),
        in_specs=[
            pl.BlockSpec((R, Q), lambda i: (i, 0)),
            pl.BlockSpec((Q, 2 * Q), lambda i: (0, 0)),
            pl.BlockSpec((2 * Q, Q), lambda i: (0, 0)),
            pl.BlockSpec((Q, Q), lambda i: (0, 0)),
            pl.BlockSpec((Q, D), lambda i: (0, 0)),
        ],
        out_specs=pl.BlockSpec((R, D), lambda i: (i, 0)),
        out_shape=jax.ShapeDtypeStruct((B, D), jnp.float32),
        compiler_params=pltpu.CompilerParams(
            dimension_semantics=("parallel",),
        ),
    )(x, w1t, w2t, w3t, wct)
    return out
